# baseline (speedup 1.0000x reference)
import jax
import jax.numpy as jnp
from jax import lax
from jax.experimental import pallas as pl
from jax.experimental.pallas import tpu as pltpu
from jax.experimental.pallas import tpu_sc as plsc

_N = 10000
_E = 320000
_NC, _NS = 2, 16
_NW = _NC * _NS
_K = 64
_CPT = 160
_CG = 32
_NG = _CPT // _CG
_EPAD = _NW * _CPT * _K
_NPAD = 10112
_RPT = _NPAD // _NS
_CW = 16
_BLK = 1000
_GRID = _N // _BLK

_f32 = jnp.float32
_sc_params = pltpu.CompilerParams(use_tc_tiling_on_sc=False)


def _core_outs(c, sl, srcs, outs_a, outs_b):
    @pl.when(c == 0)
    def _():
        for src, out in zip(srcs, outs_a):
            pltpu.sync_copy(src.at[sl], out.at[sl])

    @pl.when(c == 1)
    def _():
        for src, out in zip(srcs, outs_b):
            pltpu.sync_copy(src.at[sl], out.at[sl])


def _make_sc_segsum(width):
    mesh = plsc.VectorSubcoreMesh(core_axis_name="c", subcore_axis_name="s",
                                  num_cores=_NC, num_subcores=_NS)
    out_type = [jax.ShapeDtypeStruct((_NPAD, width), _f32)] * 2
    scratch = [pltpu.VMEM_SHARED((_NPAD, width), _f32),
               pltpu.VMEM((_CG, _K), jnp.int32),
               pltpu.VMEM((_CG, _K), jnp.int32),
               pltpu.VMEM((5, _K, width), _f32),
               pltpu.SemaphoreType.DMA,
               pltpu.SemaphoreType.DMA]

    def body(table, src_hbm, dst_hbm, zeros_hbm, out_a, out_b,
             acc, src_v, dst_v, rows, sem, sem_s):
        c = lax.axis_index("c")
        s = lax.axis_index("s")
        wid = s * _NC + c
        sl = pl.ds(s * _RPT, _RPT)
        pltpu.sync_copy(zeros_hbm, acc.at[sl])
        plsc.subcore_barrier()

        def group(g, carry):
            pltpu.sync_copy(src_hbm.at[wid, pl.ds(g * _CG, _CG)], src_v)
            pltpu.sync_copy(dst_hbm.at[wid, pl.ds(g * _CG, _CG)], dst_v)
            for p in range(4):
                pltpu.async_copy(table.at[src_v.at[p]], rows.at[p], sem)

            def chunk(j, carry2):
                b = lax.rem(j, 5)
                bn = lax.rem(j + 4, 5)
                pltpu.make_async_copy(
                    table.at[src_v.at[j]], rows.at[b], sem).wait()

                @pl.when(j >= 1)
                def _():
                    pltpu.make_async_copy(
                        rows.at[bn], acc.at[dst_v.at[j - 1]],
                        sem_s).wait()

                @pl.when(j + 4 < _CG)
                def _():
                    pltpu.async_copy(
                        table.at[src_v.at[j + 4]], rows.at[bn], sem)

                pltpu.async_copy(rows.at[b], acc.at[dst_v.at[j]], sem_s,
                                 add=True)
                return carry2

            lax.fori_loop(0, _CG, chunk, 0)
            pltpu.make_async_copy(
                rows.at[(_CG - 1) % 5], acc.at[dst_v.at[_CG - 1]],
                sem_s).wait()
            return carry

        lax.fori_loop(0, _NG, group, 0)
        plsc.subcore_barrier()
        _core_outs(c, sl, [acc], [out_a], [out_b])

    return pl.kernel(body, out_type, mesh=mesh, scratch_types=scratch,
                     compiler_params=_sc_params)


def _make_sc_count():
    mesh = plsc.VectorSubcoreMesh(core_axis_name="c", subcore_axis_name="s",
                                  num_cores=_NC, num_subcores=_NS)
    out_type = [jax.ShapeDtypeStruct((_NPAD, _CW), _f32)] * 2
    scratch = [pltpu.VMEM_SHARED((_NPAD, _CW), _f32),
               pltpu.VMEM((_CG, _K), jnp.int32),
               pltpu.VMEM((_K, _CW), _f32),
               pltpu.SemaphoreType.DMA]

    def body(dst_hbm, zc_hbm, ones_hbm, cout_a, cout_b,
             cacc, dst_v, ones_v, sem):
        c = lax.axis_index("c")
        s = lax.axis_index("s")
        wid = s * _NC + c
        sl = pl.ds(s * _RPT, _RPT)
        pltpu.sync_copy(zc_hbm, cacc.at[sl])
        pltpu.sync_copy(ones_hbm, ones_v)
        plsc.subcore_barrier()

        def group(g, carry):
            pltpu.sync_copy(dst_hbm.at[wid, pl.ds(g * _CG, _CG)], dst_v)

            def chunk(j, carry2):
                pltpu.async_copy(ones_v, cacc.at[dst_v.at[j]], sem, add=True)

                @pl.when(j > 0)
                def _():
                    pltpu.make_async_copy(
                        ones_v, cacc.at[dst_v.at[j - 1]], sem).wait()

                return carry2

            lax.fori_loop(0, _CG, chunk, 0)
            pltpu.make_async_copy(ones_v, cacc.at[dst_v.at[_CG - 1]],
                                  sem).wait()
            return carry

        lax.fori_loop(0, _NG, group, 0)
        plsc.subcore_barrier()
        _core_outs(c, sl, [cacc], [cout_a], [cout_b])

    return pl.kernel(body, out_type, mesh=mesh, scratch_types=scratch,
                     compiler_params=_sc_params)


_sc128 = _make_sc_segsum(128)
_sc64 = _make_sc_segsum(64)
_sc_cnt = _make_sc_count()


def _tc_first(x, wlT, wrT):
    def body(x_r, wl_r, wr_r, y_r, r_r):
        xb = x_r[...]
        y_r[...] = jnp.dot(xb, wl_r[...], preferred_element_type=_f32)
        r_r[...] = jnp.dot(xb, wr_r[...], preferred_element_type=_f32)

    return pl.pallas_call(
        body, grid=(_GRID,),
        in_specs=[pl.BlockSpec((_BLK, 128), lambda i: (i, 0)),
                  pl.BlockSpec((128, 128), lambda i: (0, 0)),
                  pl.BlockSpec((128, 128), lambda i: (0, 0))],
        out_specs=[pl.BlockSpec((_BLK, 128), lambda i: (i, 0))] * 2,
        out_shape=[jax.ShapeDtypeStruct((_N, 128), _f32)] * 2,
    )(x, wlT, wrT)


def _tc_mid(agg_a, agg_b, cnt_a, cnt_b, r, bl, wlT, wrT):
    win = agg_a.shape[1]
    wout = wlT.shape[1]

    def body(aA, aB, cA, cB, r_r, bl_r, wl_r, wr_r, y_r, rr_r):
        cnt = cA[...][:, :1] + cB[...][:, :1]
        inv = 1.0 / jnp.maximum(cnt, 1.0)
        h = jnp.maximum((aA[...] + aB[...]) * inv + bl_r[...] + r_r[...], 0.0)
        y_r[...] = jnp.dot(h, wl_r[...], preferred_element_type=_f32)
        rr_r[...] = jnp.dot(h, wr_r[...], preferred_element_type=_f32)

    return pl.pallas_call(
        body, grid=(_GRID,),
        in_specs=[pl.BlockSpec((_BLK, win), lambda i: (i, 0)),
                  pl.BlockSpec((_BLK, win), lambda i: (i, 0)),
                  pl.BlockSpec((_BLK, _CW), lambda i: (i, 0)),
                  pl.BlockSpec((_BLK, _CW), lambda i: (i, 0)),
                  pl.BlockSpec((_BLK, win), lambda i: (i, 0)),
                  pl.BlockSpec((1, win), lambda i: (0, 0)),
                  pl.BlockSpec((win, wout), lambda i: (0, 0)),
                  pl.BlockSpec((win, wout), lambda i: (0, 0))],
        out_specs=[pl.BlockSpec((_BLK, wout), lambda i: (i, 0))] * 2,
        out_shape=[jax.ShapeDtypeStruct((_N, wout), _f32)] * 2,
    )(agg_a, agg_b, cnt_a, cnt_b, r, bl, wlT, wrT)


def _tc_final(agg_a, agg_b, cnt_a, cnt_b, r, bl, wcT, bcp):
    win = agg_a.shape[1]

    def body(aA, aB, cA, cB, r_r, bl_r, wc_r, bc_r, o_r):
        cnt = cA[...][:, :1] + cB[...][:, :1]
        inv = 1.0 / jnp.maximum(cnt, 1.0)
        h = jnp.maximum((aA[...] + aB[...]) * inv + bl_r[...] + r_r[...], 0.0)
        o_r[...] = jnp.dot(h, wc_r[...], preferred_element_type=_f32) + bc_r[...]

    return pl.pallas_call(
        body, grid=(_GRID,),
        in_specs=[pl.BlockSpec((_BLK, win), lambda i: (i, 0)),
                  pl.BlockSpec((_BLK, win), lambda i: (i, 0)),
                  pl.BlockSpec((_BLK, _CW), lambda i: (i, 0)),
                  pl.BlockSpec((_BLK, _CW), lambda i: (i, 0)),
                  pl.BlockSpec((_BLK, win), lambda i: (i, 0)),
                  pl.BlockSpec((1, win), lambda i: (0, 0)),
                  pl.BlockSpec((win, 128), lambda i: (0, 0)),
                  pl.BlockSpec((1, 128), lambda i: (0, 0))],
        out_specs=pl.BlockSpec((_BLK, 128), lambda i: (i, 0)),
        out_shape=jax.ShapeDtypeStruct((_N, 128), _f32),
    )(agg_a, agg_b, cnt_a, cnt_b, r, bl, wcT, bcp)


def kernel(x, edge_index, Wl1, bl1, Wr1, Wl2, bl2, Wr2, Wl3, bl3, Wr3, Wc, bc):
    src = edge_index[0]
    dst = edge_index[1]
    pad = _EPAD - _E
    src_p = jnp.concatenate(
        [src, jnp.zeros((pad,), jnp.int32)]).reshape(_NW, _CPT, _K)
    dst_p = jnp.concatenate(
        [dst, jnp.full((pad,), _N, jnp.int32)]).reshape(_NW, _CPT, _K)
    z128 = jnp.zeros((_RPT, 128), _f32)
    z64 = jnp.zeros((_RPT, 64), _f32)
    zc = jnp.zeros((_RPT, _CW), _f32)
    ones = jnp.ones((_K, _CW), _f32)
    bl1r = bl1.reshape(1, -1)
    bl2r = bl2.reshape(1, -1)
    bl3r = bl3.reshape(1, -1)
    wcT = jnp.zeros((64, 128), _f32).at[:, :7].set(Wc.T)
    bcp = jnp.zeros((1, 128), _f32).at[0, :7].set(bc)

    cnt_a, cnt_b = _sc_cnt(dst_p, zc, ones)
    y1, r1 = _tc_first(x, Wl1.T, Wr1.T)
    agg_a1, agg_b1 = _sc128(y1, src_p, dst_p, z128)
    y2, r2 = _tc_mid(agg_a1, agg_b1, cnt_a, cnt_b, r1, bl1r, Wl2.T, Wr2.T)
    agg_a2, agg_b2 = _sc128(y2, src_p, dst_p, z128)
    y3, r3 = _tc_mid(agg_a2, agg_b2, cnt_a, cnt_b, r2, bl2r, Wl3.T, Wr3.T)
    agg_a3, agg_b3 = _sc64(y3, src_p, dst_p, z64)
    logits = _tc_final(agg_a3, agg_b3, cnt_a, cnt_b, r3, bl3r, wcT, bcp)
    return logits[:, :7]


# speedup vs baseline: 2.1760x; 2.1760x over previous
import jax
import jax.numpy as jnp
from jax import lax
from jax.experimental import pallas as pl
from jax.experimental.pallas import tpu as pltpu
from jax.experimental.pallas import tpu_sc as plsc

_N = 10000
_E = 320000
_NC, _NS = 2, 16
_NW = _NC * _NS
_K = 64
_CPT = 160
_CG = 32
_NG = _CPT // _CG
_EPAD = _NW * _CPT * _K
_NPAD = 10112
_RPT = _NPAD // _NS
_CW = 16
_BLK = 1000
_GRID = _N // _BLK

_f32 = jnp.float32
_sc_params = pltpu.CompilerParams(use_tc_tiling_on_sc=False)


def _core_outs(c, sl, srcs, outs_a, outs_b):
    @pl.when(c == 0)
    def _():
        for src, out in zip(srcs, outs_a):
            pltpu.sync_copy(src.at[sl], out.at[sl])

    @pl.when(c == 1)
    def _():
        for src, out in zip(srcs, outs_b):
            pltpu.sync_copy(src.at[sl], out.at[sl])


def _make_sc_segsum(halves):
    mesh = plsc.VectorSubcoreMesh(core_axis_name="c", subcore_axis_name="s",
                                  num_cores=_NC, num_subcores=_NS)
    out_type = [jax.ShapeDtypeStruct((_NPAD, 64), _f32)] * (2 * halves)
    scratch = [pltpu.VMEM_SHARED((_NPAD, 64), _f32),
               pltpu.VMEM_SHARED((_NPAD, 64), _f32),
               pltpu.VMEM((_CG, _K), jnp.int32),
               pltpu.VMEM((_CG, _K), jnp.int32),
               pltpu.VMEM((5, _K, 64), _f32),
               pltpu.SemaphoreType.DMA,
               pltpu.SemaphoreType.DMA]

    def body(*refs):
        ys = refs[:halves]
        src_hbm, dst_hbm, zeros_hbm = refs[halves:halves + 3]
        outs = refs[halves + 3:halves + 3 + 2 * halves]
        (table_sh, acc, src_v, dst_v, rows, sem, sem_s) = \
            refs[halves + 3 + 2 * halves:]
        c = lax.axis_index("c")
        s = lax.axis_index("s")
        wid = s * _NC + c
        sl = pl.ds(s * _RPT, _RPT)

        def group(g, carry):
            pltpu.sync_copy(src_hbm.at[wid, pl.ds(g * _CG, _CG)], src_v)
            pltpu.sync_copy(dst_hbm.at[wid, pl.ds(g * _CG, _CG)], dst_v)
            for p in range(4):
                pltpu.async_copy(table_sh.at[src_v.at[p]], rows.at[p], sem)

            def chunk(j, carry2):
                b = lax.rem(j, 5)
                bn = lax.rem(j + 4, 5)
                pltpu.make_async_copy(
                    table_sh.at[src_v.at[j]], rows.at[b], sem).wait()

                @pl.when(j >= 1)
                def _():
                    pltpu.make_async_copy(
                        rows.at[bn], acc.at[dst_v.at[j - 1]],
                        sem_s).wait()

                @pl.when(j + 4 < _CG)
                def _():
                    pltpu.async_copy(
                        table_sh.at[src_v.at[j + 4]], rows.at[bn], sem)

                pltpu.async_copy(rows.at[b], acc.at[dst_v.at[j]], sem_s,
                                 add=True)
                return carry2

            lax.fori_loop(0, _CG, chunk, 0)
            pltpu.make_async_copy(
                rows.at[(_CG - 1) % 5], acc.at[dst_v.at[_CG - 1]],
                sem_s).wait()
            return carry

        for h in range(halves):
            pltpu.sync_copy(ys[h].at[sl], table_sh.at[sl])
            pltpu.sync_copy(zeros_hbm, acc.at[sl])
            plsc.subcore_barrier()
            lax.fori_loop(0, _NG, group, 0)
            plsc.subcore_barrier()
            _core_outs(c, sl, [acc], [outs[2 * h]], [outs[2 * h + 1]])
            if h + 1 < halves:
                plsc.subcore_barrier()

    return pl.kernel(body, out_type, mesh=mesh, scratch_types=scratch,
                     compiler_params=_sc_params)


def _make_sc_count():
    mesh = plsc.VectorSubcoreMesh(core_axis_name="c", subcore_axis_name="s",
                                  num_cores=_NC, num_subcores=_NS)
    out_type = [jax.ShapeDtypeStruct((_NPAD, _CW), _f32)] * 2
    scratch = [pltpu.VMEM_SHARED((_NPAD, _CW), _f32),
               pltpu.VMEM((_CG, _K), jnp.int32),
               pltpu.VMEM((_K, _CW), _f32),
               pltpu.SemaphoreType.DMA]

    def body(dst_hbm, zc_hbm, ones_hbm, cout_a, cout_b,
             cacc, dst_v, ones_v, sem):
        c = lax.axis_index("c")
        s = lax.axis_index("s")
        wid = s * _NC + c
        sl = pl.ds(s * _RPT, _RPT)
        pltpu.sync_copy(zc_hbm, cacc.at[sl])
        pltpu.sync_copy(ones_hbm, ones_v)
        plsc.subcore_barrier()

        def group(g, carry):
            pltpu.sync_copy(dst_hbm.at[wid, pl.ds(g * _CG, _CG)], dst_v)

            def chunk(j, carry2):
                pltpu.async_copy(ones_v, cacc.at[dst_v.at[j]], sem, add=True)

                @pl.when(j > 0)
                def _():
                    pltpu.make_async_copy(
                        ones_v, cacc.at[dst_v.at[j - 1]], sem).wait()

                return carry2

            lax.fori_loop(0, _CG, chunk, 0)
            pltpu.make_async_copy(ones_v, cacc.at[dst_v.at[_CG - 1]],
                                  sem).wait()
            return carry

        lax.fori_loop(0, _NG, group, 0)
        plsc.subcore_barrier()
        _core_outs(c, sl, [cacc], [cout_a], [cout_b])

    return pl.kernel(body, out_type, mesh=mesh, scratch_types=scratch,
                     compiler_params=_sc_params)


_sc128 = _make_sc_segsum(2)
_sc64 = _make_sc_segsum(1)
_sc_cnt = _make_sc_count()


def _tc_first(x, wlT, wrT):
    def body(x_r, wl_r, wr_r, ylo_r, yhi_r, r_r):
        xb = x_r[...]
        y = jnp.dot(xb, wl_r[...], preferred_element_type=_f32)
        ylo_r[...] = y[:, :64]
        yhi_r[...] = y[:, 64:]
        r_r[...] = jnp.dot(xb, wr_r[...], preferred_element_type=_f32)

    return pl.pallas_call(
        body, grid=(_GRID,),
        in_specs=[pl.BlockSpec((_BLK, 128), lambda i: (i, 0)),
                  pl.BlockSpec((128, 128), lambda i: (0, 0)),
                  pl.BlockSpec((128, 128), lambda i: (0, 0))],
        out_specs=[pl.BlockSpec((_BLK, 64), lambda i: (i, 0)),
                   pl.BlockSpec((_BLK, 64), lambda i: (i, 0)),
                   pl.BlockSpec((_BLK, 128), lambda i: (i, 0))],
        out_shape=[jax.ShapeDtypeStruct((_NPAD, 64), _f32),
                   jax.ShapeDtypeStruct((_NPAD, 64), _f32),
                   jax.ShapeDtypeStruct((_N, 128), _f32)],
    )(x, wlT, wrT)


def _tc_mid(aggs, cnt_a, cnt_b, r, bl, wlT, wrT, split_y):
    win = 64 * (len(aggs) // 2)
    wout = wlT.shape[1]

    def body(*refs):
        arefs = refs[:len(aggs)]
        cA, cB, r_r, bl_r, wl_r, wr_r = refs[len(aggs):len(aggs) + 6]
        orefs = refs[len(aggs) + 6:]
        cnt = cA[...][:, :1] + cB[...][:, :1]
        inv = 1.0 / jnp.maximum(cnt, 1.0)
        agg = jnp.concatenate(
            [arefs[2 * h][...] + arefs[2 * h + 1][...]
             for h in range(len(aggs) // 2)], axis=1)
        h = jnp.maximum(agg * inv + bl_r[...] + r_r[...], 0.0)
        y = jnp.dot(h, wl_r[...], preferred_element_type=_f32)
        if split_y:
            orefs[0][...] = y[:, :64]
            orefs[1][...] = y[:, 64:]
        else:
            orefs[0][...] = y
        orefs[-1][...] = jnp.dot(h, wr_r[...], preferred_element_type=_f32)

    wrout = wrT.shape[1]
    y_specs = ([pl.BlockSpec((_BLK, 64), lambda i: (i, 0))] * 2 if split_y
               else [pl.BlockSpec((_BLK, wout), lambda i: (i, 0))])
    y_shapes = ([jax.ShapeDtypeStruct((_NPAD, 64), _f32)] * 2 if split_y
                else [jax.ShapeDtypeStruct((_NPAD, wout), _f32)])
    return pl.pallas_call(
        body, grid=(_GRID,),
        in_specs=[pl.BlockSpec((_BLK, 64), lambda i: (i, 0))] * len(aggs) +
                 [pl.BlockSpec((_BLK, _CW), lambda i: (i, 0)),
                  pl.BlockSpec((_BLK, _CW), lambda i: (i, 0)),
                  pl.BlockSpec((_BLK, win), lambda i: (i, 0)),
                  pl.BlockSpec((1, win), lambda i: (0, 0)),
                  pl.BlockSpec((win, wout), lambda i: (0, 0)),
                  pl.BlockSpec((win, wrout), lambda i: (0, 0))],
        out_specs=y_specs + [pl.BlockSpec((_BLK, wrout), lambda i: (i, 0))],
        out_shape=y_shapes + [jax.ShapeDtypeStruct((_N, wrout), _f32)],
    )(*aggs, cnt_a, cnt_b, r, bl, wlT, wrT)


def _tc_final(agg_a, agg_b, cnt_a, cnt_b, r, bl, wcT, bcp):
    win = agg_a.shape[1]

    def body(aA, aB, cA, cB, r_r, bl_r, wc_r, bc_r, o_r):
        cnt = cA[...][:, :1] + cB[...][:, :1]
        inv = 1.0 / jnp.maximum(cnt, 1.0)
        h = jnp.maximum((aA[...] + aB[...]) * inv + bl_r[...] + r_r[...], 0.0)
        o_r[...] = jnp.dot(h, wc_r[...], preferred_element_type=_f32) + bc_r[...]

    return pl.pallas_call(
        body, grid=(_GRID,),
        in_specs=[pl.BlockSpec((_BLK, win), lambda i: (i, 0)),
                  pl.BlockSpec((_BLK, win), lambda i: (i, 0)),
                  pl.BlockSpec((_BLK, _CW), lambda i: (i, 0)),
                  pl.BlockSpec((_BLK, _CW), lambda i: (i, 0)),
                  pl.BlockSpec((_BLK, win), lambda i: (i, 0)),
                  pl.BlockSpec((1, win), lambda i: (0, 0)),
                  pl.BlockSpec((win, 128), lambda i: (0, 0)),
                  pl.BlockSpec((1, 128), lambda i: (0, 0))],
        out_specs=pl.BlockSpec((_BLK, 128), lambda i: (i, 0)),
        out_shape=jax.ShapeDtypeStruct((_N, 128), _f32),
    )(agg_a, agg_b, cnt_a, cnt_b, r, bl, wcT, bcp)


def kernel(x, edge_index, Wl1, bl1, Wr1, Wl2, bl2, Wr2, Wl3, bl3, Wr3, Wc, bc):
    src = edge_index[0]
    dst = edge_index[1]
    pad = _EPAD - _E
    src_p = jnp.concatenate(
        [src, jnp.zeros((pad,), jnp.int32)]).reshape(_NW, _CPT, _K)
    dst_p = jnp.concatenate(
        [dst, jnp.full((pad,), _N, jnp.int32)]).reshape(_NW, _CPT, _K)
    z64 = jnp.zeros((_RPT, 64), _f32)
    zc = jnp.zeros((_RPT, _CW), _f32)
    ones = jnp.ones((_K, _CW), _f32)
    bl1r = bl1.reshape(1, -1)
    bl2r = bl2.reshape(1, -1)
    bl3r = bl3.reshape(1, -1)
    wcT = jnp.zeros((64, 128), _f32).at[:, :7].set(Wc.T)
    bcp = jnp.zeros((1, 128), _f32).at[0, :7].set(bc)

    cnt_a, cnt_b = _sc_cnt(dst_p, zc, ones)
    y1lo, y1hi, r1 = _tc_first(x, Wl1.T, Wr1.T)
    a1 = _sc128(y1lo, y1hi, src_p, dst_p, z64)
    y2lo, y2hi, r2 = _tc_mid(a1, cnt_a, cnt_b, r1, bl1r,
                             Wl2.T, Wr2.T, split_y=True)
    a2 = _sc128(y2lo, y2hi, src_p, dst_p, z64)
    y3, r3 = _tc_mid(a2, cnt_a, cnt_b, r2, bl2r,
                     Wl3.T, Wr3.T, split_y=False)
    agg_a3, agg_b3 = _sc64(y3, src_p, dst_p, z64)
    logits = _tc_final(agg_a3, agg_b3, cnt_a, cnt_b, r3, bl3r, wcT, bcp)
    return logits[:, :7]


# speedup vs baseline: 2.1778x; 1.0008x over previous
import jax
import jax.numpy as jnp
from jax import lax
from jax.experimental import pallas as pl
from jax.experimental.pallas import tpu as pltpu
from jax.experimental.pallas import tpu_sc as plsc

_N = 10000
_E = 320000
_NC, _NS = 2, 16
_NW = _NC * _NS
_K = 64
_CPT = 160
_CG = 32
_NG = _CPT // _CG
_EPAD = _NW * _CPT * _K
_NPAD = 10112
_RPT = _NPAD // _NS
_CW = 16
_BLK = 1000
_GRID = _N // _BLK

_f32 = jnp.float32
_sc_params = pltpu.CompilerParams(use_tc_tiling_on_sc=False)


def _core_outs(c, sl, srcs, outs_a, outs_b):
    @pl.when(c == 0)
    def _():
        for src, out in zip(srcs, outs_a):
            pltpu.sync_copy(src.at[sl], out.at[sl])

    @pl.when(c == 1)
    def _():
        for src, out in zip(srcs, outs_b):
            pltpu.sync_copy(src.at[sl], out.at[sl])


def _make_sc_segsum(halves, with_cnt=False):
    mesh = plsc.VectorSubcoreMesh(core_axis_name="c", subcore_axis_name="s",
                                  num_cores=_NC, num_subcores=_NS)
    out_type = [jax.ShapeDtypeStruct((_NPAD, 64), _f32)] * (2 * halves)
    scratch = [pltpu.VMEM_SHARED((_NPAD, 64), _f32),
               pltpu.VMEM_SHARED((_NPAD, 64), _f32),
               pltpu.VMEM((_CG, _K), jnp.int32),
               pltpu.VMEM((_CG, _K), jnp.int32),
               pltpu.VMEM((5, _K, 64), _f32),
               pltpu.SemaphoreType.DMA,
               pltpu.SemaphoreType.DMA]
    n_in = halves + 3
    if with_cnt:
        out_type += [jax.ShapeDtypeStruct((_NPAD, _CW), _f32)] * 2
        scratch += [pltpu.VMEM_SHARED((_NPAD, _CW), _f32),
                    pltpu.VMEM((_K, _CW), _f32),
                    pltpu.SemaphoreType.DMA]
        n_in += 2

    n_out = 2 * halves + (2 if with_cnt else 0)

    def body(*refs):
        ys = refs[:halves]
        src_hbm, dst_hbm, zeros_hbm = refs[halves:halves + 3]
        outs = refs[n_in:n_in + 2 * halves]
        (table_sh, acc, src_v, dst_v, rows, sem, sem_s) = \
            refs[n_in + n_out:n_in + n_out + 7]
        if with_cnt:
            zc_hbm, ones_hbm = refs[halves + 3:n_in]
            cout_a, cout_b = refs[n_in + 2 * halves:n_in + n_out]
            cacc, ones_v, sem_c = refs[n_in + n_out + 7:]
        c = lax.axis_index("c")
        s = lax.axis_index("s")
        wid = s * _NC + c
        sl = pl.ds(s * _RPT, _RPT)

        def make_group(do_cnt):
            def group(g, carry):
                pltpu.sync_copy(src_hbm.at[wid, pl.ds(g * _CG, _CG)], src_v)
                pltpu.sync_copy(dst_hbm.at[wid, pl.ds(g * _CG, _CG)], dst_v)
                for p in range(4):
                    pltpu.async_copy(table_sh.at[src_v.at[p]], rows.at[p],
                                     sem)

                def chunk(j, carry2):
                    b = lax.rem(j, 5)
                    bn = lax.rem(j + 4, 5)
                    pltpu.make_async_copy(
                        table_sh.at[src_v.at[j]], rows.at[b], sem).wait()

                    @pl.when(j >= 1)
                    def _():
                        pltpu.make_async_copy(
                            rows.at[bn], acc.at[dst_v.at[j - 1]],
                            sem_s).wait()

                    @pl.when(j + 4 < _CG)
                    def _():
                        pltpu.async_copy(
                            table_sh.at[src_v.at[j + 4]], rows.at[bn], sem)

                    pltpu.async_copy(rows.at[b], acc.at[dst_v.at[j]], sem_s,
                                     add=True)
                    if do_cnt:
                        pltpu.async_copy(ones_v, cacc.at[dst_v.at[j]],
                                         sem_c, add=True)

                        @pl.when(j >= 1)
                        def _():
                            pltpu.make_async_copy(
                                ones_v, cacc.at[dst_v.at[j - 1]],
                                sem_c).wait()

                    return carry2

                lax.fori_loop(0, _CG, chunk, 0)
                pltpu.make_async_copy(
                    rows.at[(_CG - 1) % 5], acc.at[dst_v.at[_CG - 1]],
                    sem_s).wait()
                if do_cnt:
                    pltpu.make_async_copy(
                        ones_v, cacc.at[dst_v.at[_CG - 1]], sem_c).wait()
                return carry
            return group

        for h in range(halves):
            pltpu.sync_copy(ys[h].at[sl], table_sh.at[sl])
            pltpu.sync_copy(zeros_hbm, acc.at[sl])
            if with_cnt and h == 0:
                pltpu.sync_copy(zc_hbm, cacc.at[sl])
                pltpu.sync_copy(ones_hbm, ones_v)
            plsc.subcore_barrier()
            lax.fori_loop(0, _NG, make_group(with_cnt and h == 0), 0)
            plsc.subcore_barrier()
            _core_outs(c, sl, [acc], [outs[2 * h]], [outs[2 * h + 1]])
            if with_cnt and h == 0:
                _core_outs(c, sl, [cacc], [cout_a], [cout_b])
            if h + 1 < halves:
                plsc.subcore_barrier()

    return pl.kernel(body, out_type, mesh=mesh, scratch_types=scratch,
                     compiler_params=_sc_params)


_sc128c = _make_sc_segsum(2, with_cnt=True)
_sc128 = _make_sc_segsum(2)
_sc64 = _make_sc_segsum(1)


def _tc_first(x, wlT, wrT):
    def body(x_r, wl_r, wr_r, ylo_r, yhi_r, r_r):
        xb = x_r[...]
        y = jnp.dot(xb, wl_r[...], preferred_element_type=_f32)
        ylo_r[...] = y[:, :64]
        yhi_r[...] = y[:, 64:]
        r_r[...] = jnp.dot(xb, wr_r[...], preferred_element_type=_f32)

    return pl.pallas_call(
        body, grid=(_GRID,),
        in_specs=[pl.BlockSpec((_BLK, 128), lambda i: (i, 0)),
                  pl.BlockSpec((128, 128), lambda i: (0, 0)),
                  pl.BlockSpec((128, 128), lambda i: (0, 0))],
        out_specs=[pl.BlockSpec((_BLK, 64), lambda i: (i, 0)),
                   pl.BlockSpec((_BLK, 64), lambda i: (i, 0)),
                   pl.BlockSpec((_BLK, 128), lambda i: (i, 0))],
        out_shape=[jax.ShapeDtypeStruct((_NPAD, 64), _f32),
                   jax.ShapeDtypeStruct((_NPAD, 64), _f32),
                   jax.ShapeDtypeStruct((_N, 128), _f32)],
    )(x, wlT, wrT)


def _tc_mid(aggs, cnt_a, cnt_b, r, bl, wlT, wrT, split_y):
    win = 64 * (len(aggs) // 2)
    wout = wlT.shape[1]

    def body(*refs):
        arefs = refs[:len(aggs)]
        cA, cB, r_r, bl_r, wl_r, wr_r = refs[len(aggs):len(aggs) + 6]
        orefs = refs[len(aggs) + 6:]
        cnt = cA[...][:, :1] + cB[...][:, :1]
        inv = 1.0 / jnp.maximum(cnt, 1.0)
        agg = jnp.concatenate(
            [arefs[2 * h][...] + arefs[2 * h + 1][...]
             for h in range(len(aggs) // 2)], axis=1)
        h = jnp.maximum(agg * inv + bl_r[...] + r_r[...], 0.0)
        y = jnp.dot(h, wl_r[...], preferred_element_type=_f32)
        if split_y:
            orefs[0][...] = y[:, :64]
            orefs[1][...] = y[:, 64:]
        else:
            orefs[0][...] = y
        orefs[-1][...] = jnp.dot(h, wr_r[...], preferred_element_type=_f32)

    wrout = wrT.shape[1]
    y_specs = ([pl.BlockSpec((_BLK, 64), lambda i: (i, 0))] * 2 if split_y
               else [pl.BlockSpec((_BLK, wout), lambda i: (i, 0))])
    y_shapes = ([jax.ShapeDtypeStruct((_NPAD, 64), _f32)] * 2 if split_y
                else [jax.ShapeDtypeStruct((_NPAD, wout), _f32)])
    return pl.pallas_call(
        body, grid=(_GRID,),
        in_specs=[pl.BlockSpec((_BLK, 64), lambda i: (i, 0))] * len(aggs) +
                 [pl.BlockSpec((_BLK, _CW), lambda i: (i, 0)),
                  pl.BlockSpec((_BLK, _CW), lambda i: (i, 0)),
                  pl.BlockSpec((_BLK, win), lambda i: (i, 0)),
                  pl.BlockSpec((1, win), lambda i: (0, 0)),
                  pl.BlockSpec((win, wout), lambda i: (0, 0)),
                  pl.BlockSpec((win, wrout), lambda i: (0, 0))],
        out_specs=y_specs + [pl.BlockSpec((_BLK, wrout), lambda i: (i, 0))],
        out_shape=y_shapes + [jax.ShapeDtypeStruct((_N, wrout), _f32)],
    )(*aggs, cnt_a, cnt_b, r, bl, wlT, wrT)


def _tc_final(agg_a, agg_b, cnt_a, cnt_b, r, bl, wcT, bcp):
    win = agg_a.shape[1]

    def body(aA, aB, cA, cB, r_r, bl_r, wc_r, bc_r, o_r):
        cnt = cA[...][:, :1] + cB[...][:, :1]
        inv = 1.0 / jnp.maximum(cnt, 1.0)
        h = jnp.maximum((aA[...] + aB[...]) * inv + bl_r[...] + r_r[...], 0.0)
        o_r[...] = jnp.dot(h, wc_r[...], preferred_element_type=_f32) + bc_r[...]

    return pl.pallas_call(
        body, grid=(_GRID,),
        in_specs=[pl.BlockSpec((_BLK, win), lambda i: (i, 0)),
                  pl.BlockSpec((_BLK, win), lambda i: (i, 0)),
                  pl.BlockSpec((_BLK, _CW), lambda i: (i, 0)),
                  pl.BlockSpec((_BLK, _CW), lambda i: (i, 0)),
                  pl.BlockSpec((_BLK, win), lambda i: (i, 0)),
                  pl.BlockSpec((1, win), lambda i: (0, 0)),
                  pl.BlockSpec((win, 128), lambda i: (0, 0)),
                  pl.BlockSpec((1, 128), lambda i: (0, 0))],
        out_specs=pl.BlockSpec((_BLK, 128), lambda i: (i, 0)),
        out_shape=jax.ShapeDtypeStruct((_N, 128), _f32),
    )(agg_a, agg_b, cnt_a, cnt_b, r, bl, wcT, bcp)


def kernel(x, edge_index, Wl1, bl1, Wr1, Wl2, bl2, Wr2, Wl3, bl3, Wr3, Wc, bc):
    src = edge_index[0]
    dst = edge_index[1]
    pad = _EPAD - _E
    src_p = jnp.concatenate(
        [src, jnp.zeros((pad,), jnp.int32)]).reshape(_NW, _CPT, _K)
    dst_p = jnp.concatenate(
        [dst, jnp.full((pad,), _N, jnp.int32)]).reshape(_NW, _CPT, _K)
    z64 = jnp.zeros((_RPT, 64), _f32)
    zc = jnp.zeros((_RPT, _CW), _f32)
    ones = jnp.ones((_K, _CW), _f32)
    bl1r = bl1.reshape(1, -1)
    bl2r = bl2.reshape(1, -1)
    bl3r = bl3.reshape(1, -1)
    wcT = jnp.zeros((64, 128), _f32).at[:, :7].set(Wc.T)
    bcp = jnp.zeros((1, 128), _f32).at[0, :7].set(bc)

    y1lo, y1hi, r1 = _tc_first(x, Wl1.T, Wr1.T)
    *a1, cnt_a, cnt_b = _sc128c(y1lo, y1hi, src_p, dst_p, z64, zc, ones)
    y2lo, y2hi, r2 = _tc_mid(a1, cnt_a, cnt_b, r1, bl1r,
                             Wl2.T, Wr2.T, split_y=True)
    a2 = _sc128(y2lo, y2hi, src_p, dst_p, z64)
    y3, r3 = _tc_mid(a2, cnt_a, cnt_b, r2, bl2r,
                     Wl3.T, Wr3.T, split_y=False)
    agg_a3, agg_b3 = _sc64(y3, src_p, dst_p, z64)
    logits = _tc_final(agg_a3, agg_b3, cnt_a, cnt_b, r3, bl3r, wcT, bcp)
    return logits[:, :7]


# speedup vs baseline: 2.2978x; 1.0551x over previous
import jax
import jax.numpy as jnp
from jax import lax
from jax.experimental import pallas as pl
from jax.experimental.pallas import tpu as pltpu
from jax.experimental.pallas import tpu_sc as plsc

_N = 10000
_E = 320000
_NC, _NS = 2, 16
_K = 64
_CPT = 320
_CG = 32
_NG = _CPT // _CG
_EPAD = _NS * _CPT * _K
_NPAD = 10112
_RPT = _NPAD // _NS
_CW = 16
_BLK = 1000
_GRID = _N // _BLK

_f32 = jnp.float32
_sc_params = pltpu.CompilerParams(use_tc_tiling_on_sc=False)


def _make_sc_segsum(w, with_cnt=False):
    mesh = plsc.VectorSubcoreMesh(core_axis_name="c", subcore_axis_name="s",
                                  num_cores=_NC, num_subcores=_NS)
    out_type = [jax.ShapeDtypeStruct((_NPAD, w), _f32)] * 2
    scratch = [pltpu.VMEM_SHARED((_NPAD, w), _f32),
               pltpu.VMEM_SHARED((_NPAD, w), _f32),
               pltpu.VMEM((_CG, _K), jnp.int32),
               pltpu.VMEM((_CG, _K), jnp.int32),
               pltpu.VMEM((5, _K, w), _f32),
               pltpu.SemaphoreType.DMA,
               pltpu.SemaphoreType.DMA]
    n_in = 5 + (2 if with_cnt else 0)
    n_out = 2 + (2 if with_cnt else 0)
    if with_cnt:
        out_type += [jax.ShapeDtypeStruct((_NPAD, _CW), _f32)] * 2
        scratch += [pltpu.VMEM_SHARED((_NPAD, _CW), _f32),
                    pltpu.VMEM((_K, _CW), _f32),
                    pltpu.SemaphoreType.DMA]

    def body(*refs):
        y0, y1, src_hbm, dst_hbm, zeros_hbm = refs[:5]
        out0, out1 = refs[n_in:n_in + 2]
        (table_sh, acc, src_v, dst_v, rows, sem, sem_s) = \
            refs[n_in + n_out:n_in + n_out + 7]
        if with_cnt:
            zc_hbm, ones_hbm = refs[5:7]
            cout_a, cout_b = refs[n_in + 2:n_in + 4]
            cacc, ones_v, sem_c = refs[n_in + n_out + 7:]
        c = lax.axis_index("c")
        s = lax.axis_index("s")
        sl = pl.ds(s * _RPT, _RPT)

        @pl.when(c == 0)
        def _():
            pltpu.sync_copy(y0.at[sl], table_sh.at[sl])

        @pl.when(c == 1)
        def _():
            pltpu.sync_copy(y1.at[sl], table_sh.at[sl])

        pltpu.sync_copy(zeros_hbm, acc.at[sl])
        if with_cnt:
            pltpu.sync_copy(zc_hbm, cacc.at[sl])
            pltpu.sync_copy(ones_hbm, ones_v)
        plsc.subcore_barrier()

        def group(g, carry):
            pltpu.sync_copy(src_hbm.at[s, pl.ds(g * _CG, _CG)], src_v)
            pltpu.sync_copy(dst_hbm.at[s, pl.ds(g * _CG, _CG)], dst_v)
            if with_cnt:
                cp = jnp.equal(c == 0, g < _NG // 2)
            for p in range(4):
                pltpu.async_copy(table_sh.at[src_v.at[p]], rows.at[p], sem)

            def chunk(j, carry2):
                b = lax.rem(j, 5)
                bn = lax.rem(j + 4, 5)
                pltpu.make_async_copy(
                    table_sh.at[src_v.at[j]], rows.at[b], sem).wait()

                @pl.when(j >= 1)
                def _():
                    pltpu.make_async_copy(
                        rows.at[bn], acc.at[dst_v.at[j - 1]],
                        sem_s).wait()

                @pl.when(j + 4 < _CG)
                def _():
                    pltpu.async_copy(
                        table_sh.at[src_v.at[j + 4]], rows.at[bn], sem)

                pltpu.async_copy(rows.at[b], acc.at[dst_v.at[j]], sem_s,
                                 add=True)
                if with_cnt:
                    @pl.when(cp)
                    def _():
                        pltpu.async_copy(ones_v, cacc.at[dst_v.at[j]],
                                         sem_c, add=True)

                    @pl.when(jnp.logical_and(cp, j >= 1))
                    def _():
                        pltpu.make_async_copy(
                            ones_v, cacc.at[dst_v.at[j - 1]],
                            sem_c).wait()

                return carry2

            lax.fori_loop(0, _CG, chunk, 0)
            pltpu.make_async_copy(
                rows.at[(_CG - 1) % 5], acc.at[dst_v.at[_CG - 1]],
                sem_s).wait()
            if with_cnt:
                @pl.when(cp)
                def _():
                    pltpu.make_async_copy(
                        ones_v, cacc.at[dst_v.at[_CG - 1]], sem_c).wait()
            return carry

        lax.fori_loop(0, _NG, group, 0)
        plsc.subcore_barrier()

        @pl.when(c == 0)
        def _():
            pltpu.sync_copy(acc.at[sl], out0.at[sl])
            if with_cnt:
                pltpu.sync_copy(cacc.at[sl], cout_a.at[sl])

        @pl.when(c == 1)
        def _():
            pltpu.sync_copy(acc.at[sl], out1.at[sl])
            if with_cnt:
                pltpu.sync_copy(cacc.at[sl], cout_b.at[sl])

    return pl.kernel(body, out_type, mesh=mesh, scratch_types=scratch,
                     compiler_params=_sc_params)


_sc64c = _make_sc_segsum(64, with_cnt=True)
_sc64 = _make_sc_segsum(64)
_sc32 = _make_sc_segsum(32)


def _tc_first(x, wlT, wrT):
    def body(x_r, wl_r, wr_r, ylo_r, yhi_r, r_r):
        xb = x_r[...]
        y = jnp.dot(xb, wl_r[...], preferred_element_type=_f32)
        ylo_r[...] = y[:, :64]
        yhi_r[...] = y[:, 64:]
        r_r[...] = jnp.dot(xb, wr_r[...], preferred_element_type=_f32)

    return pl.pallas_call(
        body, grid=(_GRID,),
        in_specs=[pl.BlockSpec((_BLK, 128), lambda i: (i, 0)),
                  pl.BlockSpec((128, 128), lambda i: (0, 0)),
                  pl.BlockSpec((128, 128), lambda i: (0, 0))],
        out_specs=[pl.BlockSpec((_BLK, 64), lambda i: (i, 0)),
                   pl.BlockSpec((_BLK, 64), lambda i: (i, 0)),
                   pl.BlockSpec((_BLK, 128), lambda i: (i, 0))],
        out_shape=[jax.ShapeDtypeStruct((_NPAD, 64), _f32),
                   jax.ShapeDtypeStruct((_NPAD, 64), _f32),
                   jax.ShapeDtypeStruct((_N, 128), _f32)],
    )(x, wlT, wrT)


def _tc_mid(agg0, agg1, cnt_a, cnt_b, r, bl, wlT, wrT):
    wh = agg0.shape[1]
    win = 2 * wh
    wout = wlT.shape[1]
    wrout = wrT.shape[1]
    wyh = wout // 2

    def body(a0, a1, cA, cB, r_r, bl_r, wl_r, wr_r, ylo_r, yhi_r, rr_r):
        cnt = cA[...][:, :1] + cB[...][:, :1]
        inv = 1.0 / jnp.maximum(cnt, 1.0)
        agg = jnp.concatenate([a0[...], a1[...]], axis=1)
        h = jnp.maximum(agg * inv + bl_r[...] + r_r[...], 0.0)
        y = jnp.dot(h, wl_r[...], preferred_element_type=_f32)
        ylo_r[...] = y[:, :wyh]
        yhi_r[...] = y[:, wyh:]
        rr_r[...] = jnp.dot(h, wr_r[...], preferred_element_type=_f32)

    return pl.pallas_call(
        body, grid=(_GRID,),
        in_specs=[pl.BlockSpec((_BLK, wh), lambda i: (i, 0)),
                  pl.BlockSpec((_BLK, wh), lambda i: (i, 0)),
                  pl.BlockSpec((_BLK, _CW), lambda i: (i, 0)),
                  pl.BlockSpec((_BLK, _CW), lambda i: (i, 0)),
                  pl.BlockSpec((_BLK, win), lambda i: (i, 0)),
                  pl.BlockSpec((1, win), lambda i: (0, 0)),
                  pl.BlockSpec((win, wout), lambda i: (0, 0)),
                  pl.BlockSpec((win, wrout), lambda i: (0, 0))],
        out_specs=[pl.BlockSpec((_BLK, wyh), lambda i: (i, 0)),
                   pl.BlockSpec((_BLK, wyh), lambda i: (i, 0)),
                   pl.BlockSpec((_BLK, wrout), lambda i: (i, 0))],
        out_shape=[jax.ShapeDtypeStruct((_NPAD, wyh), _f32),
                   jax.ShapeDtypeStruct((_NPAD, wyh), _f32),
                   jax.ShapeDtypeStruct((_N, wrout), _f32)],
    )(agg0, agg1, cnt_a, cnt_b, r, bl, wlT, wrT)


def _tc_final(agg0, agg1, cnt_a, cnt_b, r, bl, wcT, bcp):
    wh = agg0.shape[1]
    win = 2 * wh

    def body(a0, a1, cA, cB, r_r, bl_r, wc_r, bc_r, o_r):
        cnt = cA[...][:, :1] + cB[...][:, :1]
        inv = 1.0 / jnp.maximum(cnt, 1.0)
        agg = jnp.concatenate([a0[...], a1[...]], axis=1)
        h = jnp.maximum(agg * inv + bl_r[...] + r_r[...], 0.0)
        o_r[...] = jnp.dot(h, wc_r[...], preferred_element_type=_f32) + bc_r[...]

    return pl.pallas_call(
        body, grid=(_GRID,),
        in_specs=[pl.BlockSpec((_BLK, wh), lambda i: (i, 0)),
                  pl.BlockSpec((_BLK, wh), lambda i: (i, 0)),
                  pl.BlockSpec((_BLK, _CW), lambda i: (i, 0)),
                  pl.BlockSpec((_BLK, _CW), lambda i: (i, 0)),
                  pl.BlockSpec((_BLK, win), lambda i: (i, 0)),
                  pl.BlockSpec((1, win), lambda i: (0, 0)),
                  pl.BlockSpec((win, 128), lambda i: (0, 0)),
                  pl.BlockSpec((1, 128), lambda i: (0, 0))],
        out_specs=pl.BlockSpec((_BLK, 128), lambda i: (i, 0)),
        out_shape=jax.ShapeDtypeStruct((_N, 128), _f32),
    )(agg0, agg1, cnt_a, cnt_b, r, bl, wcT, bcp)


def kernel(x, edge_index, Wl1, bl1, Wr1, Wl2, bl2, Wr2, Wl3, bl3, Wr3, Wc, bc):
    src = edge_index[0]
    dst = edge_index[1]
    pad = _EPAD - _E
    src_p = jnp.concatenate(
        [src, jnp.zeros((pad,), jnp.int32)]).reshape(_NS, _CPT, _K)
    dst_p = jnp.concatenate(
        [dst, jnp.full((pad,), _N, jnp.int32)]).reshape(_NS, _CPT, _K)
    z64 = jnp.zeros((_RPT, 64), _f32)
    z32 = jnp.zeros((_RPT, 32), _f32)
    zc = jnp.zeros((_RPT, _CW), _f32)
    ones = jnp.ones((_K, _CW), _f32)
    bl1r = bl1.reshape(1, -1)
    bl2r = bl2.reshape(1, -1)
    bl3r = bl3.reshape(1, -1)
    wcT = jnp.zeros((64, 128), _f32).at[:, :7].set(Wc.T)
    bcp = jnp.zeros((1, 128), _f32).at[0, :7].set(bc)

    y1lo, y1hi, r1 = _tc_first(x, Wl1.T, Wr1.T)
    a1lo, a1hi, cnt_a, cnt_b = _sc64c(y1lo, y1hi, src_p, dst_p, z64, zc, ones)
    y2lo, y2hi, r2 = _tc_mid(a1lo, a1hi, cnt_a, cnt_b, r1, bl1r,
                             Wl2.T, Wr2.T)
    a2lo, a2hi = _sc64(y2lo, y2hi, src_p, dst_p, z64)
    y3lo, y3hi, r3 = _tc_mid(a2lo, a2hi, cnt_a, cnt_b, r2, bl2r,
                             Wl3.T, Wr3.T)
    a3lo, a3hi = _sc32(y3lo, y3hi, src_p, dst_p, z32)
    logits = _tc_final(a3lo, a3hi, cnt_a, cnt_b, r3, bl3r, wcT, bcp)
    return logits[:, :7]


# speedup vs baseline: 2.3969x; 1.0431x over previous
import jax
import jax.numpy as jnp
from jax import lax
from jax.experimental import pallas as pl
from jax.experimental.pallas import tpu as pltpu
from jax.experimental.pallas import tpu_sc as plsc

_N = 10000
_E = 320000
_NC, _NS = 2, 16
_K = 64
_CPT = 320
_CG = 64
_NG = _CPT // _CG
_EPAD = _NS * _CPT * _K
_NPAD = 10112
_RPT = _NPAD // _NS
_CW = 16
_BLK = 1000
_GRID = _N // _BLK

_f32 = jnp.float32
_sc_params = pltpu.CompilerParams(use_tc_tiling_on_sc=False)


def _make_sc_segsum(w, with_cnt=False):
    mesh = plsc.VectorSubcoreMesh(core_axis_name="c", subcore_axis_name="s",
                                  num_cores=_NC, num_subcores=_NS)
    out_type = [jax.ShapeDtypeStruct((_NPAD, w), _f32)] * 2
    scratch = [pltpu.VMEM_SHARED((_NPAD, w), _f32),
               pltpu.VMEM_SHARED((_NPAD, w), _f32),
               pltpu.VMEM((_CG, _K), jnp.int32),
               pltpu.VMEM((_CG, _K), jnp.int32),
               pltpu.VMEM((5, _K, w), _f32),
               pltpu.SemaphoreType.DMA,
               pltpu.SemaphoreType.DMA]
    n_in = 5 + (2 if with_cnt else 0)
    n_out = 2 + (2 if with_cnt else 0)
    if with_cnt:
        out_type += [jax.ShapeDtypeStruct((_NPAD, _CW), _f32)] * 2
        scratch += [pltpu.VMEM_SHARED((_NPAD, _CW), _f32),
                    pltpu.VMEM((_K, _CW), _f32),
                    pltpu.SemaphoreType.DMA]

    def body(*refs):
        y0, y1, src_hbm, dst_hbm, zeros_hbm = refs[:5]
        out0, out1 = refs[n_in:n_in + 2]
        (table_sh, acc, src_v, dst_v, rows, sem, sem_s) = \
            refs[n_in + n_out:n_in + n_out + 7]
        if with_cnt:
            zc_hbm, ones_hbm = refs[5:7]
            cout_a, cout_b = refs[n_in + 2:n_in + 4]
            cacc, ones_v, sem_c = refs[n_in + n_out + 7:]
        c = lax.axis_index("c")
        s = lax.axis_index("s")
        sl = pl.ds(s * _RPT, _RPT)

        @pl.when(c == 0)
        def _():
            pltpu.sync_copy(y0.at[sl], table_sh.at[sl])

        @pl.when(c == 1)
        def _():
            pltpu.sync_copy(y1.at[sl], table_sh.at[sl])

        pltpu.sync_copy(zeros_hbm, acc.at[sl])
        if with_cnt:
            pltpu.sync_copy(zc_hbm, cacc.at[sl])
            pltpu.sync_copy(ones_hbm, ones_v)
        plsc.subcore_barrier()

        def group(g, carry):
            pltpu.sync_copy(src_hbm.at[s, pl.ds(g * _CG, _CG)], src_v)
            pltpu.sync_copy(dst_hbm.at[s, pl.ds(g * _CG, _CG)], dst_v)
            if with_cnt:
                cp = jnp.equal(c == 0, g < _NG // 2)
            for p in range(4):
                pltpu.async_copy(table_sh.at[src_v.at[p]], rows.at[p], sem)

            def chunk(j, carry2):
                b = lax.rem(j, 5)
                bn = lax.rem(j + 4, 5)
                pltpu.make_async_copy(
                    table_sh.at[src_v.at[j]], rows.at[b], sem).wait()

                @pl.when(j >= 1)
                def _():
                    pltpu.make_async_copy(
                        rows.at[bn], acc.at[dst_v.at[j - 1]],
                        sem_s).wait()

                @pl.when(j + 4 < _CG)
                def _():
                    pltpu.async_copy(
                        table_sh.at[src_v.at[j + 4]], rows.at[bn], sem)

                pltpu.async_copy(rows.at[b], acc.at[dst_v.at[j]], sem_s,
                                 add=True)
                if with_cnt:
                    @pl.when(cp)
                    def _():
                        pltpu.async_copy(ones_v, cacc.at[dst_v.at[j]],
                                         sem_c, add=True)

                    @pl.when(jnp.logical_and(cp, j >= 1))
                    def _():
                        pltpu.make_async_copy(
                            ones_v, cacc.at[dst_v.at[j - 1]],
                            sem_c).wait()

                return carry2

            lax.fori_loop(0, _CG, chunk, 0)
            pltpu.make_async_copy(
                rows.at[(_CG - 1) % 5], acc.at[dst_v.at[_CG - 1]],
                sem_s).wait()
            if with_cnt:
                @pl.when(cp)
                def _():
                    pltpu.make_async_copy(
                        ones_v, cacc.at[dst_v.at[_CG - 1]], sem_c).wait()
            return carry

        lax.fori_loop(0, _NG, group, 0)
        plsc.subcore_barrier()

        @pl.when(c == 0)
        def _():
            pltpu.sync_copy(acc.at[sl], out0.at[sl])
            if with_cnt:
                pltpu.sync_copy(cacc.at[sl], cout_a.at[sl])

        @pl.when(c == 1)
        def _():
            pltpu.sync_copy(acc.at[sl], out1.at[sl])
            if with_cnt:
                pltpu.sync_copy(cacc.at[sl], cout_b.at[sl])

    return pl.kernel(body, out_type, mesh=mesh, scratch_types=scratch,
                     compiler_params=_sc_params)


_sc64c = _make_sc_segsum(64, with_cnt=True)
_sc64 = _make_sc_segsum(64)
_sc32 = _make_sc_segsum(32)


def _tc_first(x, wlT, wrT):
    def body(x_r, wl_r, wr_r, ylo_r, yhi_r, r_r):
        xb = x_r[...]
        y = jnp.dot(xb, wl_r[...], preferred_element_type=_f32)
        ylo_r[...] = y[:, :64]
        yhi_r[...] = y[:, 64:]
        r_r[...] = jnp.dot(xb, wr_r[...], preferred_element_type=_f32)

    return pl.pallas_call(
        body, grid=(_GRID,),
        in_specs=[pl.BlockSpec((_BLK, 128), lambda i: (i, 0)),
                  pl.BlockSpec((128, 128), lambda i: (0, 0)),
                  pl.BlockSpec((128, 128), lambda i: (0, 0))],
        out_specs=[pl.BlockSpec((_BLK, 64), lambda i: (i, 0)),
                   pl.BlockSpec((_BLK, 64), lambda i: (i, 0)),
                   pl.BlockSpec((_BLK, 128), lambda i: (i, 0))],
        out_shape=[jax.ShapeDtypeStruct((_NPAD, 64), _f32),
                   jax.ShapeDtypeStruct((_NPAD, 64), _f32),
                   jax.ShapeDtypeStruct((_N, 128), _f32)],
    )(x, wlT, wrT)


def _tc_mid(agg0, agg1, cnt_a, cnt_b, r, bl, wlT, wrT):
    wh = agg0.shape[1]
    win = 2 * wh
    wout = wlT.shape[1]
    wrout = wrT.shape[1]
    wyh = wout // 2

    def body(a0, a1, cA, cB, r_r, bl_r, wl_r, wr_r, ylo_r, yhi_r, rr_r):
        cnt = cA[...][:, :1] + cB[...][:, :1]
        inv = 1.0 / jnp.maximum(cnt, 1.0)
        agg = jnp.concatenate([a0[...], a1[...]], axis=1)
        h = jnp.maximum(agg * inv + bl_r[...] + r_r[...], 0.0)
        y = jnp.dot(h, wl_r[...], preferred_element_type=_f32)
        ylo_r[...] = y[:, :wyh]
        yhi_r[...] = y[:, wyh:]
        rr_r[...] = jnp.dot(h, wr_r[...], preferred_element_type=_f32)

    return pl.pallas_call(
        body, grid=(_GRID,),
        in_specs=[pl.BlockSpec((_BLK, wh), lambda i: (i, 0)),
                  pl.BlockSpec((_BLK, wh), lambda i: (i, 0)),
                  pl.BlockSpec((_BLK, _CW), lambda i: (i, 0)),
                  pl.BlockSpec((_BLK, _CW), lambda i: (i, 0)),
                  pl.BlockSpec((_BLK, win), lambda i: (i, 0)),
                  pl.BlockSpec((1, win), lambda i: (0, 0)),
                  pl.BlockSpec((win, wout), lambda i: (0, 0)),
                  pl.BlockSpec((win, wrout), lambda i: (0, 0))],
        out_specs=[pl.BlockSpec((_BLK, wyh), lambda i: (i, 0)),
                   pl.BlockSpec((_BLK, wyh), lambda i: (i, 0)),
                   pl.BlockSpec((_BLK, wrout), lambda i: (i, 0))],
        out_shape=[jax.ShapeDtypeStruct((_NPAD, wyh), _f32),
                   jax.ShapeDtypeStruct((_NPAD, wyh), _f32),
                   jax.ShapeDtypeStruct((_N, wrout), _f32)],
    )(agg0, agg1, cnt_a, cnt_b, r, bl, wlT, wrT)


def _tc_final(agg0, agg1, cnt_a, cnt_b, r, bl, wcT, bcp):
    wh = agg0.shape[1]
    win = 2 * wh

    def body(a0, a1, cA, cB, r_r, bl_r, wc_r, bc_r, o_r):
        cnt = cA[...][:, :1] + cB[...][:, :1]
        inv = 1.0 / jnp.maximum(cnt, 1.0)
        agg = jnp.concatenate([a0[...], a1[...]], axis=1)
        h = jnp.maximum(agg * inv + bl_r[...] + r_r[...], 0.0)
        o_r[...] = jnp.dot(h, wc_r[...], preferred_element_type=_f32) + bc_r[...]

    return pl.pallas_call(
        body, grid=(_GRID,),
        in_specs=[pl.BlockSpec((_BLK, wh), lambda i: (i, 0)),
                  pl.BlockSpec((_BLK, wh), lambda i: (i, 0)),
                  pl.BlockSpec((_BLK, _CW), lambda i: (i, 0)),
                  pl.BlockSpec((_BLK, _CW), lambda i: (i, 0)),
                  pl.BlockSpec((_BLK, win), lambda i: (i, 0)),
                  pl.BlockSpec((1, win), lambda i: (0, 0)),
                  pl.BlockSpec((win, 128), lambda i: (0, 0)),
                  pl.BlockSpec((1, 128), lambda i: (0, 0))],
        out_specs=pl.BlockSpec((_BLK, 128), lambda i: (i, 0)),
        out_shape=jax.ShapeDtypeStruct((_N, 128), _f32),
    )(agg0, agg1, cnt_a, cnt_b, r, bl, wcT, bcp)


def kernel(x, edge_index, Wl1, bl1, Wr1, Wl2, bl2, Wr2, Wl3, bl3, Wr3, Wc, bc):
    src = edge_index[0]
    dst = edge_index[1]
    pad = _EPAD - _E
    src_p = jnp.concatenate(
        [src, jnp.zeros((pad,), jnp.int32)]).reshape(_NS, _CPT, _K)
    dst_p = jnp.concatenate(
        [dst, jnp.full((pad,), _N, jnp.int32)]).reshape(_NS, _CPT, _K)
    z64 = jnp.zeros((_RPT, 64), _f32)
    z32 = jnp.zeros((_RPT, 32), _f32)
    zc = jnp.zeros((_RPT, _CW), _f32)
    ones = jnp.ones((_K, _CW), _f32)
    bl1r = bl1.reshape(1, -1)
    bl2r = bl2.reshape(1, -1)
    bl3r = bl3.reshape(1, -1)
    wcT = jnp.zeros((64, 128), _f32).at[:, :7].set(Wc.T)
    bcp = jnp.zeros((1, 128), _f32).at[0, :7].set(bc)

    y1lo, y1hi, r1 = _tc_first(x, Wl1.T, Wr1.T)
    a1lo, a1hi, cnt_a, cnt_b = _sc64c(y1lo, y1hi, src_p, dst_p, z64, zc, ones)
    y2lo, y2hi, r2 = _tc_mid(a1lo, a1hi, cnt_a, cnt_b, r1, bl1r,
                             Wl2.T, Wr2.T)
    a2lo, a2hi = _sc64(y2lo, y2hi, src_p, dst_p, z64)
    y3lo, y3hi, r3 = _tc_mid(a2lo, a2hi, cnt_a, cnt_b, r2, bl2r,
                             Wl3.T, Wr3.T)
    a3lo, a3hi = _sc32(y3lo, y3hi, src_p, dst_p, z32)
    logits = _tc_final(a3lo, a3hi, cnt_a, cnt_b, r3, bl3r, wcT, bcp)
    return logits[:, :7]
